# baseline (speedup 1.0000x reference)
import jax
import jax.numpy as jnp
from jax.experimental import pallas as pl

_NUM_BUCKETS = 1024.0
_PACK = 8
_BLOCK_ROWS = 5000


def _lsh_block_kernel(x_ref, p_ref, b_ref, o_ref):
    h = jax.lax.dot_general(
        x_ref[...], p_ref[...],
        dimension_numbers=(((1,), (0,)), ((), ())),
        preferred_element_type=jnp.float32,
    )
    h = h + b_ref[...]
    o_ref[...] = jnp.floor(h * (1.0 / _NUM_BUCKETS)).astype(jnp.int32)


@jax.jit
def kernel(x, projections, biases):
    n, emb = x.shape
    num_hashes = projections.shape[0]
    n_packed = n // _PACK
    xp = x.reshape(n_packed, _PACK * emb)

    eye = jnp.eye(_PACK, dtype=x.dtype)
    p_big = jnp.einsum("ij,hd->idjh", eye, projections).reshape(
        _PACK * emb, _PACK * num_hashes)
    b_big = jnp.tile(biases, _PACK).reshape(1, _PACK * num_hashes)

    grid = (pl.cdiv(n_packed, _BLOCK_ROWS),)
    out = pl.pallas_call(
        _lsh_block_kernel,
        grid=grid,
        in_specs=[
            pl.BlockSpec((_BLOCK_ROWS, _PACK * emb), lambda i: (i, 0)),
            pl.BlockSpec((_PACK * emb, _PACK * num_hashes), lambda i: (0, 0)),
            pl.BlockSpec((1, _PACK * num_hashes), lambda i: (0, 0)),
        ],
        out_specs=pl.BlockSpec((_BLOCK_ROWS, _PACK * num_hashes),
                               lambda i: (i, 0)),
        out_shape=jax.ShapeDtypeStruct((n_packed, _PACK * num_hashes),
                                       jnp.int32),
    )(xp, p_big, b_big)
    return out.reshape(n, num_hashes)


# speedup vs baseline: 11.1775x; 11.1775x over previous
import jax
import jax.numpy as jnp
from jax.experimental import pallas as pl

_NUM_BUCKETS = 1024.0
_BLOCK_C = 16384


def _lsh_block_kernel(xt_ref, p_ref, b_ref, o_ref):
    h = jax.lax.dot_general(
        p_ref[...], xt_ref[...],
        dimension_numbers=(((1,), (0,)), ((), ())),
        preferred_element_type=jnp.float32,
    )
    h = h + b_ref[...]
    o_ref[...] = jnp.floor(h * (1.0 / _NUM_BUCKETS)).astype(jnp.int32)


@jax.jit
def kernel(x, projections, biases):
    n, emb = x.shape
    num_hashes = projections.shape[0]
    xt = x.T
    grid = (pl.cdiv(n, _BLOCK_C),)
    out_t = pl.pallas_call(
        _lsh_block_kernel,
        grid=grid,
        in_specs=[
            pl.BlockSpec((emb, _BLOCK_C), lambda i: (0, i)),
            pl.BlockSpec((num_hashes, emb), lambda i: (0, 0)),
            pl.BlockSpec((num_hashes, 1), lambda i: (0, 0)),
        ],
        out_specs=pl.BlockSpec((num_hashes, _BLOCK_C), lambda i: (0, i)),
        out_shape=jax.ShapeDtypeStruct((num_hashes, n), jnp.int32),
    )(xt, projections, biases.reshape(num_hashes, 1))
    return out_t.T


# speedup vs baseline: 12.0175x; 1.0751x over previous
import jax
import jax.numpy as jnp
from jax.experimental import pallas as pl

_NUM_BUCKETS = 1024.0
_BLOCK_C = 32768


def _lsh_block_kernel(xt_ref, p_ref, b_ref, o_ref):
    h = jax.lax.dot_general(
        p_ref[...], xt_ref[...],
        dimension_numbers=(((1,), (0,)), ((), ())),
        preferred_element_type=jnp.float32,
    )
    h = h + b_ref[...]
    o_ref[...] = jnp.floor(h * (1.0 / _NUM_BUCKETS)).astype(jnp.int32)


@jax.jit
def kernel(x, projections, biases):
    n, emb = x.shape
    num_hashes = projections.shape[0]
    xt = x.T
    grid = (pl.cdiv(n, _BLOCK_C),)
    out_t = pl.pallas_call(
        _lsh_block_kernel,
        grid=grid,
        in_specs=[
            pl.BlockSpec((emb, _BLOCK_C), lambda i: (0, i)),
            pl.BlockSpec((num_hashes, emb), lambda i: (0, 0)),
            pl.BlockSpec((num_hashes, 1), lambda i: (0, 0)),
        ],
        out_specs=pl.BlockSpec((num_hashes, _BLOCK_C), lambda i: (0, i)),
        out_shape=jax.ShapeDtypeStruct((num_hashes, n), jnp.int32),
    )(xt, projections, biases.reshape(num_hashes, 1))
    return out_t.T
